# baseline (speedup 1.0000x reference)
import functools

import jax
import jax.numpy as jnp
from jax import lax
from jax.experimental import pallas as pl
from jax.experimental.pallas import tpu as pltpu
from jax.experimental.pallas import tpu_sc as plsc

VOCAB = 100000
E = 128
H = 128
B = 1024
L = 200
NG = 4 * H

_NC = 2
_NS = 16
_NW = _NC * _NS
_ROWS = B * L
_PER_W = _ROWS // _NW
_CH = 128
_NCH = _PER_W // _CH


@functools.cache
def _make_embed_gather():
    @functools.partial(
        pl.kernel,
        mesh=plsc.VectorSubcoreMesh(core_axis_name="c", subcore_axis_name="s"),
        out_type=jax.ShapeDtypeStruct((_ROWS, E), jnp.float32),
        scratch_types=[
            pltpu.VMEM((_PER_W,), jnp.int32),
            pltpu.VMEM((_CH, E), jnp.float32),
            pltpu.VMEM((_CH, E), jnp.float32),
            pltpu.SemaphoreType.DMA,
            pltpu.SemaphoreType.DMA,
        ],
    )
    def _embed_gather(emb_hbm, idx_hbm, out_hbm, idx_v, rows0, rows1, s0, s1):
        wid = lax.axis_index("s") * _NC + lax.axis_index("c")
        base = wid * _PER_W
        pltpu.sync_copy(idx_hbm.at[pl.ds(base, _PER_W)], idx_v)

        def gather(i, rows, sem):
            pltpu.async_copy(emb_hbm.at[idx_v.at[pl.ds(i * _CH, _CH)]],
                             rows, sem)

        def drain(rows, sem):
            pltpu.make_async_copy(emb_hbm.at[pl.ds(0, _CH)], rows, sem).wait()

        gather(0, rows0, s0)

        def body(j, carry):
            i = 2 * j
            drain(rows0, s0)
            gather(i + 1, rows1, s1)
            pltpu.sync_copy(rows0, out_hbm.at[pl.ds(base + i * _CH, _CH)])
            drain(rows1, s1)

            @pl.when(i + 2 < _NCH)
            def _():
                gather(i + 2, rows0, s0)

            pltpu.sync_copy(rows1, out_hbm.at[pl.ds(base + (i + 1) * _CH, _CH)])
            return carry

        lax.fori_loop(0, _NCH // 2, body, 0)

    return _embed_gather


def _lstm_body(zeros_ref, xf_ref, xb_ref, wf_ref, wr_ref,
               bf_ref, br_ref, wfcf_ref, wfcr_ref, bfc_ref, out_ref,
               hf, cf, hb, cb, zsem):
    t = pl.program_id(0)

    @pl.when(t == 0)
    def _init():
        for dst in (hf, cf, hb, cb):
            pltpu.async_copy(zeros_ref, dst, zsem)
        for dst in (hf, cf, hb, cb):
            pltpu.make_async_copy(zeros_ref, dst, zsem).wait()

    def step(x, h, c, w, b):
        xh = jnp.concatenate([x, h], axis=-1)
        gates = jnp.dot(xh, w, preferred_element_type=jnp.float32) + b
        t_i = jnp.tanh(gates[:, 0 * H:1 * H])
        f_g = jax.nn.sigmoid(gates[:, 1 * H:2 * H])
        g_t = jnp.tanh(gates[:, 2 * H:3 * H])
        t_o = jnp.tanh(gates[:, 3 * H:4 * H])
        c_new = f_g * c + 0.5 * (t_i * g_t + g_t)
        ct = jnp.tanh(c_new)
        h_new = 0.5 * (t_o * ct + ct)
        return h_new, c_new

    hf_new, cf_new = step(xf_ref[0], hf[...], cf[...],
                          wf_ref[...], bf_ref[...])
    hb_new, cb_new = step(xb_ref[0], hb[...], cb[...],
                          wr_ref[...], br_ref[...])
    hf[...] = hf_new
    cf[...] = cf_new
    hb[...] = hb_new
    cb[...] = cb_new

    @pl.when(t == L - 1)
    def _finish():
        out_ref[...] = (
            jnp.dot(hf_new, wfcf_ref[...], preferred_element_type=jnp.float32)
            + jnp.dot(hb_new, wfcr_ref[...], preferred_element_type=jnp.float32)
            + bfc_ref[0, 0])


_lstm_call = pl.pallas_call(
    _lstm_body,
    grid=(L,),
    in_specs=[
        pl.BlockSpec(memory_space=pl.ANY),
        pl.BlockSpec((1, B, E), lambda t: (t, 0, 0)),
        pl.BlockSpec((1, B, E), lambda t: (L - 1 - t, 0, 0)),
        pl.BlockSpec((E + H, NG), lambda t: (0, 0)),
        pl.BlockSpec((E + H, NG), lambda t: (0, 0)),
        pl.BlockSpec((1, NG), lambda t: (0, 0)),
        pl.BlockSpec((1, NG), lambda t: (0, 0)),
        pl.BlockSpec((H, E), lambda t: (0, 0)),
        pl.BlockSpec((H, E), lambda t: (0, 0)),
        pl.BlockSpec(memory_space=pltpu.SMEM),
    ],
    out_specs=pl.BlockSpec((B, E), lambda t: (0, 0)),
    out_shape=jax.ShapeDtypeStruct((B, E), jnp.float32),
    scratch_shapes=[pltpu.VMEM((B, H), jnp.float32)] * 4
    + [pltpu.SemaphoreType.DMA],
    compiler_params=pltpu.CompilerParams(
        dimension_semantics=("arbitrary",)),
)


def _prep_w(Wih, Whh, bih, bhh):
    w = jnp.concatenate([Wih.T, Whh.T], axis=0)
    b = (bih + bhh).reshape(1, NG)
    scale = jnp.concatenate([
        jnp.full((H,), 0.5, jnp.float32),
        jnp.ones((H,), jnp.float32),
        jnp.ones((H,), jnp.float32),
        jnp.full((H,), 0.5, jnp.float32),
    ])
    return w * scale, b * scale


def kernel(x, emb, Wih_f, Whh_f, bih_f, bhh_f,
           Wih_r, Whh_r, bih_r, bhh_r, W_fc, b_fc):
    idx = jnp.transpose(x).reshape(-1).astype(jnp.int32)
    xs = _make_embed_gather()(emb, idx).reshape(L, B, E)
    wf, bf = _prep_w(Wih_f, Whh_f, bih_f, bhh_f)
    wr, br = _prep_w(Wih_r, Whh_r, bih_r, bhh_r)
    wfcf = jnp.broadcast_to(W_fc[0, :H].reshape(H, 1), (H, E))
    wfcr = jnp.broadcast_to(W_fc[0, H:].reshape(H, 1), (H, E))
    zeros = jnp.zeros((B, H), jnp.float32)
    out = _lstm_call(zeros, xs, xs, wf, wr, bf, br, wfcf, wfcr,
                     b_fc.reshape(1, 1))
    return out[:, 0]


# speedup vs baseline: 1.1157x; 1.1157x over previous
import functools

import jax
import jax.numpy as jnp
from jax import lax
from jax.experimental import pallas as pl
from jax.experimental.pallas import tpu as pltpu
from jax.experimental.pallas import tpu_sc as plsc

VOCAB = 100000
E = 128
H = 128
B = 1024
L = 200
NG = 4 * H

_NC = 2
_NS = 16
_NW = _NC * _NS
_ROWS = B * L
_PER_W = _ROWS // _NW
_CH = 128
_NCH = _PER_W // _CH


@functools.cache
def _make_embed_gather():
    @functools.partial(
        pl.kernel,
        mesh=plsc.VectorSubcoreMesh(core_axis_name="c", subcore_axis_name="s"),
        out_type=jax.ShapeDtypeStruct((_ROWS, E), jnp.float32),
        scratch_types=[
            pltpu.VMEM((_PER_W,), jnp.int32),
            pltpu.VMEM((_CH, E), jnp.float32),
            pltpu.VMEM((_CH, E), jnp.float32),
            pltpu.SemaphoreType.DMA,
            pltpu.SemaphoreType.DMA,
        ],
    )
    def _embed_gather(emb_hbm, idx_hbm, out_hbm, idx_v, rows0, rows1, s0, s1):
        wid = lax.axis_index("s") * _NC + lax.axis_index("c")
        base = wid * _PER_W
        pltpu.sync_copy(idx_hbm.at[pl.ds(base, _PER_W)], idx_v)

        def gather(i, rows, sem):
            pltpu.async_copy(emb_hbm.at[idx_v.at[pl.ds(i * _CH, _CH)]],
                             rows, sem)

        def drain(rows, sem):
            pltpu.make_async_copy(emb_hbm.at[pl.ds(0, _CH)], rows, sem).wait()

        gather(0, rows0, s0)

        def body(j, carry):
            i = 2 * j
            drain(rows0, s0)
            gather(i + 1, rows1, s1)
            pltpu.sync_copy(rows0, out_hbm.at[pl.ds(base + i * _CH, _CH)])
            drain(rows1, s1)

            @pl.when(i + 2 < _NCH)
            def _():
                gather(i + 2, rows0, s0)

            pltpu.sync_copy(rows1, out_hbm.at[pl.ds(base + (i + 1) * _CH, _CH)])
            return carry

        lax.fori_loop(0, _NCH // 2, body, 0)

    return _embed_gather


def _lstm_body(zeros_ref, xf_ref, xb_ref, wf_ref, wr_ref,
               bf_ref, br_ref, wfcf_ref, wfcr_ref, bfc_ref, out_ref,
               hf, cf, hb, cb, zsem):
    t = pl.program_id(0)

    @pl.when(t == 0)
    def _init():
        for dst in (hf, cf, hb, cb):
            pltpu.async_copy(zeros_ref, dst, zsem)
        for dst in (hf, cf, hb, cb):
            pltpu.make_async_copy(zeros_ref, dst, zsem).wait()

    def step(x, h, c, w, b):
        xh = jnp.concatenate([x, h], axis=-1)
        gates = jnp.dot(xh, w, preferred_element_type=jnp.float32) + b
        t_i = jnp.tanh(gates[:, 0 * H:1 * H])
        f_g = jax.nn.sigmoid(gates[:, 1 * H:2 * H])
        g_t = jnp.tanh(gates[:, 2 * H:3 * H])
        t_o = jnp.tanh(gates[:, 3 * H:4 * H])
        c_new = f_g * c + 0.5 * (t_i * g_t + g_t)
        ct = jnp.tanh(c_new)
        h_new = 0.5 * (t_o * ct + ct)
        return h_new, c_new

    hf1, cf1 = step(xf_ref[0], hf[...], cf[...], wf_ref[...], bf_ref[...])
    hb1, cb1 = step(xb_ref[1], hb[...], cb[...], wr_ref[...], br_ref[...])
    hf_new, cf_new = step(xf_ref[1], hf1, cf1, wf_ref[...], bf_ref[...])
    hb_new, cb_new = step(xb_ref[0], hb1, cb1, wr_ref[...], br_ref[...])
    hf[...] = hf_new
    cf[...] = cf_new
    hb[...] = hb_new
    cb[...] = cb_new

    @pl.when(t == L // 2 - 1)
    def _finish():
        out_ref[...] = (
            jnp.dot(hf_new, wfcf_ref[...], preferred_element_type=jnp.float32)
            + jnp.dot(hb_new, wfcr_ref[...], preferred_element_type=jnp.float32)
            + bfc_ref[0, 0])


_lstm_call = pl.pallas_call(
    _lstm_body,
    grid=(L // 2,),
    in_specs=[
        pl.BlockSpec(memory_space=pl.ANY),
        pl.BlockSpec((2, B, E), lambda t: (t, 0, 0)),
        pl.BlockSpec((2, B, E), lambda t: (L // 2 - 1 - t, 0, 0)),
        pl.BlockSpec((E + H, NG), lambda t: (0, 0)),
        pl.BlockSpec((E + H, NG), lambda t: (0, 0)),
        pl.BlockSpec((1, NG), lambda t: (0, 0)),
        pl.BlockSpec((1, NG), lambda t: (0, 0)),
        pl.BlockSpec((H, E), lambda t: (0, 0)),
        pl.BlockSpec((H, E), lambda t: (0, 0)),
        pl.BlockSpec(memory_space=pltpu.SMEM),
    ],
    out_specs=pl.BlockSpec((B, E), lambda t: (0, 0)),
    out_shape=jax.ShapeDtypeStruct((B, E), jnp.float32),
    scratch_shapes=[pltpu.VMEM((B, H), jnp.float32)] * 4
    + [pltpu.SemaphoreType.DMA],
    compiler_params=pltpu.CompilerParams(
        dimension_semantics=("arbitrary",)),
)


def _prep_w(Wih, Whh, bih, bhh):
    w = jnp.concatenate([Wih.T, Whh.T], axis=0)
    b = (bih + bhh).reshape(1, NG)
    scale = jnp.concatenate([
        jnp.full((H,), 0.5, jnp.float32),
        jnp.ones((H,), jnp.float32),
        jnp.ones((H,), jnp.float32),
        jnp.full((H,), 0.5, jnp.float32),
    ])
    return w * scale, b * scale


def kernel(x, emb, Wih_f, Whh_f, bih_f, bhh_f,
           Wih_r, Whh_r, bih_r, bhh_r, W_fc, b_fc):
    idx = jnp.transpose(x).reshape(-1).astype(jnp.int32)
    xs = _make_embed_gather()(emb, idx).reshape(L, B, E)
    wf, bf = _prep_w(Wih_f, Whh_f, bih_f, bhh_f)
    wr, br = _prep_w(Wih_r, Whh_r, bih_r, bhh_r)
    wfcf = jnp.broadcast_to(W_fc[0, :H].reshape(H, 1), (H, E))
    wfcr = jnp.broadcast_to(W_fc[0, H:].reshape(H, 1), (H, E))
    zeros = jnp.zeros((B, H), jnp.float32)
    out = _lstm_call(zeros, xs, xs, wf, wr, bf, br, wfcf, wfcr,
                     b_fc.reshape(1, 1))
    return out[:, 0]


# speedup vs baseline: 1.1803x; 1.0580x over previous
import functools

import jax
import jax.numpy as jnp
from jax import lax
from jax.experimental import pallas as pl
from jax.experimental.pallas import tpu as pltpu
from jax.experimental.pallas import tpu_sc as plsc

VOCAB = 100000
E = 128
H = 128
B = 1024
L = 200
NG = 4 * H

_NC = 2
_NS = 16
_NW = _NC * _NS
_ROWS = B * L
_PER_W = _ROWS // _NW
_CH = 128
_NCH = _PER_W // _CH


@functools.cache
def _make_embed_gather():
    @functools.partial(
        pl.kernel,
        mesh=plsc.VectorSubcoreMesh(core_axis_name="c", subcore_axis_name="s"),
        out_type=jax.ShapeDtypeStruct((_ROWS, E), jnp.float32),
        scratch_types=[
            pltpu.VMEM((_PER_W,), jnp.int32),
            pltpu.VMEM((_CH, E), jnp.float32),
            pltpu.VMEM((_CH, E), jnp.float32),
            pltpu.SemaphoreType.DMA,
            pltpu.SemaphoreType.DMA,
        ],
    )
    def _embed_gather(emb_hbm, idx_hbm, out_hbm, idx_v, rows0, rows1, s0, s1):
        wid = lax.axis_index("s") * _NC + lax.axis_index("c")
        base = wid * _PER_W
        pltpu.sync_copy(idx_hbm.at[pl.ds(base, _PER_W)], idx_v)

        def gather(i, rows, sem):
            pltpu.async_copy(emb_hbm.at[idx_v.at[pl.ds(i * _CH, _CH)]],
                             rows, sem)

        def drain(rows, sem):
            pltpu.make_async_copy(emb_hbm.at[pl.ds(0, _CH)], rows, sem).wait()

        gather(0, rows0, s0)

        def body(j, carry):
            i = 2 * j
            drain(rows0, s0)
            gather(i + 1, rows1, s1)
            pltpu.sync_copy(rows0, out_hbm.at[pl.ds(base + i * _CH, _CH)])
            drain(rows1, s1)

            @pl.when(i + 2 < _NCH)
            def _():
                gather(i + 2, rows0, s0)

            pltpu.sync_copy(rows1, out_hbm.at[pl.ds(base + (i + 1) * _CH, _CH)])
            return carry

        lax.fori_loop(0, _NCH // 2, body, 0)

    return _embed_gather


def _lstm_body(zeros_ref, xf_ref, xb_ref, wf_ref, wr_ref,
               bf_ref, br_ref, wfcf_ref, wfcr_ref, bfc_ref, out_ref,
               hf, cf, hb, cb, zsem):
    t = pl.program_id(0)

    @pl.when(t == 0)
    def _init():
        for dst in (hf, cf, hb, cb):
            pltpu.async_copy(zeros_ref, dst, zsem)
        for dst in (hf, cf, hb, cb):
            pltpu.make_async_copy(zeros_ref, dst, zsem).wait()

    def step(x, h, c, w, b):
        xh = jnp.concatenate([x, h], axis=-1)
        gates = jnp.dot(xh, w, preferred_element_type=jnp.float32) + b
        t_i = jnp.tanh(gates[:, 0 * H:1 * H])
        f_g = jax.nn.sigmoid(gates[:, 1 * H:2 * H])
        g_t = jnp.tanh(gates[:, 2 * H:3 * H])
        t_o = jnp.tanh(gates[:, 3 * H:4 * H])
        c_new = f_g * c + 0.5 * (t_i * g_t + g_t)
        ct = jnp.tanh(c_new)
        h_new = 0.5 * (t_o * ct + ct)
        return h_new, c_new

    hfc, cfc = hf[...], cf[...]
    hbc, cbc = hb[...], cb[...]
    for k in range(4):
        hfc, cfc = step(xf_ref[k], hfc, cfc, wf_ref[...], bf_ref[...])
        hbc, cbc = step(xb_ref[3 - k], hbc, cbc, wr_ref[...], br_ref[...])
    hf_new, cf_new, hb_new, cb_new = hfc, cfc, hbc, cbc
    hf[...] = hf_new
    cf[...] = cf_new
    hb[...] = hb_new
    cb[...] = cb_new

    @pl.when(t == L // 4 - 1)
    def _finish():
        out_ref[...] = (
            jnp.dot(hf_new, wfcf_ref[...], preferred_element_type=jnp.float32)
            + jnp.dot(hb_new, wfcr_ref[...], preferred_element_type=jnp.float32)
            + bfc_ref[0, 0])


_lstm_call = pl.pallas_call(
    _lstm_body,
    grid=(L // 4,),
    in_specs=[
        pl.BlockSpec(memory_space=pl.ANY),
        pl.BlockSpec((4, B, E), lambda t: (t, 0, 0)),
        pl.BlockSpec((4, B, E), lambda t: (L // 4 - 1 - t, 0, 0)),
        pl.BlockSpec((E + H, NG), lambda t: (0, 0)),
        pl.BlockSpec((E + H, NG), lambda t: (0, 0)),
        pl.BlockSpec((1, NG), lambda t: (0, 0)),
        pl.BlockSpec((1, NG), lambda t: (0, 0)),
        pl.BlockSpec((H, E), lambda t: (0, 0)),
        pl.BlockSpec((H, E), lambda t: (0, 0)),
        pl.BlockSpec(memory_space=pltpu.SMEM),
    ],
    out_specs=pl.BlockSpec((B, E), lambda t: (0, 0)),
    out_shape=jax.ShapeDtypeStruct((B, E), jnp.float32),
    scratch_shapes=[pltpu.VMEM((B, H), jnp.float32)] * 4
    + [pltpu.SemaphoreType.DMA],
    compiler_params=pltpu.CompilerParams(
        dimension_semantics=("arbitrary",)),
)


def _prep_w(Wih, Whh, bih, bhh):
    w = jnp.concatenate([Wih.T, Whh.T], axis=0)
    b = (bih + bhh).reshape(1, NG)
    scale = jnp.concatenate([
        jnp.full((H,), 0.5, jnp.float32),
        jnp.ones((H,), jnp.float32),
        jnp.ones((H,), jnp.float32),
        jnp.full((H,), 0.5, jnp.float32),
    ])
    return w * scale, b * scale


def kernel(x, emb, Wih_f, Whh_f, bih_f, bhh_f,
           Wih_r, Whh_r, bih_r, bhh_r, W_fc, b_fc):
    idx = jnp.transpose(x).reshape(-1).astype(jnp.int32)
    xs = _make_embed_gather()(emb, idx).reshape(L, B, E)
    wf, bf = _prep_w(Wih_f, Whh_f, bih_f, bhh_f)
    wr, br = _prep_w(Wih_r, Whh_r, bih_r, bhh_r)
    wfcf = jnp.broadcast_to(W_fc[0, :H].reshape(H, 1), (H, E))
    wfcr = jnp.broadcast_to(W_fc[0, H:].reshape(H, 1), (H, E))
    zeros = jnp.zeros((B, H), jnp.float32)
    out = _lstm_call(zeros, xs, xs, wf, wr, bf, br, wfcf, wfcr,
                     b_fc.reshape(1, 1))
    return out[:, 0]


# speedup vs baseline: 1.2010x; 1.0175x over previous
import functools

import jax
import jax.numpy as jnp
from jax import lax
from jax.experimental import pallas as pl
from jax.experimental.pallas import tpu as pltpu
from jax.experimental.pallas import tpu_sc as plsc

VOCAB = 100000
E = 128
H = 128
B = 1024
L = 200
NG = 4 * H

_NC = 2
_NS = 16
_NW = _NC * _NS
_ROWS = B * L
_PER_W = _ROWS // _NW
_CH = 128
_NCH = _PER_W // _CH


@functools.cache
def _make_embed_gather():
    @functools.partial(
        pl.kernel,
        mesh=plsc.VectorSubcoreMesh(core_axis_name="c", subcore_axis_name="s"),
        out_type=jax.ShapeDtypeStruct((_ROWS, E), jnp.float32),
        scratch_types=[
            pltpu.VMEM((_PER_W,), jnp.int32),
            pltpu.VMEM((_CH, E), jnp.float32),
            pltpu.VMEM((_CH, E), jnp.float32),
            pltpu.SemaphoreType.DMA,
            pltpu.SemaphoreType.DMA,
        ],
    )
    def _embed_gather(emb_hbm, idx_hbm, out_hbm, idx_v, rows0, rows1, s0, s1):
        wid = lax.axis_index("s") * _NC + lax.axis_index("c")
        base = wid * _PER_W
        pltpu.sync_copy(idx_hbm.at[pl.ds(base, _PER_W)], idx_v)

        def gather(i, rows, sem):
            pltpu.async_copy(emb_hbm.at[idx_v.at[pl.ds(i * _CH, _CH)]],
                             rows, sem)

        def drain(rows, sem):
            pltpu.make_async_copy(emb_hbm.at[pl.ds(0, _CH)], rows, sem).wait()

        gather(0, rows0, s0)

        def body(j, carry):
            i = 2 * j
            drain(rows0, s0)
            gather(i + 1, rows1, s1)
            pltpu.sync_copy(rows0, out_hbm.at[pl.ds(base + i * _CH, _CH)])
            drain(rows1, s1)

            @pl.when(i + 2 < _NCH)
            def _():
                gather(i + 2, rows0, s0)

            pltpu.sync_copy(rows1, out_hbm.at[pl.ds(base + (i + 1) * _CH, _CH)])
            return carry

        lax.fori_loop(0, _NCH // 2, body, 0)

    return _embed_gather


def _lstm_body(zeros_ref, xf_ref, xb_ref, wf_ref, wr_ref,
               bf_ref, br_ref, wfcf_ref, wfcr_ref, bfc_ref, out_ref,
               hf, cf, hb, cb, zsem):
    t = pl.program_id(0)

    @pl.when(t == 0)
    def _init():
        for dst in (hf, cf, hb, cb):
            pltpu.async_copy(zeros_ref, dst, zsem)
        for dst in (hf, cf, hb, cb):
            pltpu.make_async_copy(zeros_ref, dst, zsem).wait()

    def step(x, h, c, w, b):
        xh = jnp.concatenate([x, h], axis=-1)
        gates = jnp.dot(xh, w, preferred_element_type=jnp.float32) + b
        t_i = jnp.tanh(gates[:, 0 * H:1 * H])
        f_g = jax.nn.sigmoid(gates[:, 1 * H:2 * H])
        g_t = jnp.tanh(gates[:, 2 * H:3 * H])
        t_o = jnp.tanh(gates[:, 3 * H:4 * H])
        c_new = f_g * c + 0.5 * (t_i * g_t + g_t)
        ct = jnp.tanh(c_new)
        h_new = 0.5 * (t_o * ct + ct)
        return h_new, c_new

    hfc, cfc = hf[...], cf[...]
    hbc, cbc = hb[...], cb[...]
    for k in range(8):
        hfc, cfc = step(xf_ref[k], hfc, cfc, wf_ref[...], bf_ref[...])
        hbc, cbc = step(xb_ref[7 - k], hbc, cbc, wr_ref[...], br_ref[...])
    hf_new, cf_new, hb_new, cb_new = hfc, cfc, hbc, cbc
    hf[...] = hf_new
    cf[...] = cf_new
    hb[...] = hb_new
    cb[...] = cb_new

    @pl.when(t == L // 8 - 1)
    def _finish():
        out_ref[...] = (
            jnp.dot(hf_new, wfcf_ref[...], preferred_element_type=jnp.float32)
            + jnp.dot(hb_new, wfcr_ref[...], preferred_element_type=jnp.float32)
            + bfc_ref[0, 0])


_lstm_call = pl.pallas_call(
    _lstm_body,
    grid=(L // 8,),
    in_specs=[
        pl.BlockSpec(memory_space=pl.ANY),
        pl.BlockSpec((8, B, E), lambda t: (t, 0, 0)),
        pl.BlockSpec((8, B, E), lambda t: (L // 8 - 1 - t, 0, 0)),
        pl.BlockSpec((E + H, NG), lambda t: (0, 0)),
        pl.BlockSpec((E + H, NG), lambda t: (0, 0)),
        pl.BlockSpec((1, NG), lambda t: (0, 0)),
        pl.BlockSpec((1, NG), lambda t: (0, 0)),
        pl.BlockSpec((H, E), lambda t: (0, 0)),
        pl.BlockSpec((H, E), lambda t: (0, 0)),
        pl.BlockSpec(memory_space=pltpu.SMEM),
    ],
    out_specs=pl.BlockSpec((B, E), lambda t: (0, 0)),
    out_shape=jax.ShapeDtypeStruct((B, E), jnp.float32),
    scratch_shapes=[pltpu.VMEM((B, H), jnp.float32)] * 4
    + [pltpu.SemaphoreType.DMA],
    compiler_params=pltpu.CompilerParams(
        dimension_semantics=("arbitrary",)),
)


def _prep_w(Wih, Whh, bih, bhh):
    w = jnp.concatenate([Wih.T, Whh.T], axis=0)
    b = (bih + bhh).reshape(1, NG)
    scale = jnp.concatenate([
        jnp.full((H,), 0.5, jnp.float32),
        jnp.ones((H,), jnp.float32),
        jnp.ones((H,), jnp.float32),
        jnp.full((H,), 0.5, jnp.float32),
    ])
    return w * scale, b * scale


def kernel(x, emb, Wih_f, Whh_f, bih_f, bhh_f,
           Wih_r, Whh_r, bih_r, bhh_r, W_fc, b_fc):
    idx = jnp.transpose(x).reshape(-1).astype(jnp.int32)
    xs = _make_embed_gather()(emb, idx).reshape(L, B, E)
    wf, bf = _prep_w(Wih_f, Whh_f, bih_f, bhh_f)
    wr, br = _prep_w(Wih_r, Whh_r, bih_r, bhh_r)
    wfcf = jnp.broadcast_to(W_fc[0, :H].reshape(H, 1), (H, E))
    wfcr = jnp.broadcast_to(W_fc[0, H:].reshape(H, 1), (H, E))
    zeros = jnp.zeros((B, H), jnp.float32)
    out = _lstm_call(zeros, xs, xs, wf, wr, bf, br, wfcf, wfcr,
                     b_fc.reshape(1, 1))
    return out[:, 0]


# speedup vs baseline: 1.2729x; 1.0599x over previous
import functools

import jax
import jax.numpy as jnp
from jax import lax
from jax.experimental import pallas as pl
from jax.experimental.pallas import tpu as pltpu
from jax.experimental.pallas import tpu_sc as plsc

VOCAB = 100000
E = 128
H = 128
B = 1024
L = 200
NG = 4 * H

_NC = 2
_NS = 16
_NW = _NC * _NS
_ROWS = B * L
_PER_W = _ROWS // _NW
_CH = 256
_NCH = _PER_W // _CH


@functools.cache
def _make_embed_gather():
    @functools.partial(
        pl.kernel,
        mesh=plsc.VectorSubcoreMesh(core_axis_name="c", subcore_axis_name="s"),
        out_type=jax.ShapeDtypeStruct((_ROWS, E), jnp.float32),
        scratch_types=[
            pltpu.VMEM((_PER_W,), jnp.int32),
            pltpu.VMEM((_CH, E), jnp.float32),
            pltpu.VMEM((_CH, E), jnp.float32),
            pltpu.SemaphoreType.DMA,
            pltpu.SemaphoreType.DMA,
        ],
    )
    def _embed_gather(emb_hbm, idx_hbm, out_hbm, idx_v, rows0, rows1, s0, s1):
        wid = lax.axis_index("s") * _NC + lax.axis_index("c")
        base = wid * _PER_W
        pltpu.sync_copy(idx_hbm.at[pl.ds(base, _PER_W)], idx_v)

        def gather(i, rows, sem):
            pltpu.async_copy(emb_hbm.at[idx_v.at[pl.ds(i * _CH, _CH)]],
                             rows, sem)

        def drain(rows, sem):
            pltpu.make_async_copy(emb_hbm.at[pl.ds(0, _CH)], rows, sem).wait()

        gather(0, rows0, s0)

        def body(j, carry):
            i = 2 * j
            drain(rows0, s0)
            gather(i + 1, rows1, s1)
            pltpu.sync_copy(rows0, out_hbm.at[pl.ds(base + i * _CH, _CH)])
            drain(rows1, s1)

            @pl.when(i + 2 < _NCH)
            def _():
                gather(i + 2, rows0, s0)

            pltpu.sync_copy(rows1, out_hbm.at[pl.ds(base + (i + 1) * _CH, _CH)])
            return carry

        lax.fori_loop(0, _NCH // 2, body, 0)

        if _NCH % 2:
            i = _NCH - 1
            drain(rows0, s0)
            pltpu.sync_copy(rows0, out_hbm.at[pl.ds(base + i * _CH, _CH)])

    return _embed_gather


def _lstm_body(zeros_ref, xf_ref, xb_ref, wf_ref, wr_ref,
               bf_ref, br_ref, wfcf_ref, wfcr_ref, bfc_ref, out_ref,
               hf, cf, hb, cb, zsem):
    t = pl.program_id(0)

    @pl.when(t == 0)
    def _init():
        for dst in (hf, cf, hb, cb):
            pltpu.async_copy(zeros_ref, dst, zsem)
        for dst in (hf, cf, hb, cb):
            pltpu.make_async_copy(zeros_ref, dst, zsem).wait()

    def step(x, h, c, w, b):
        xh = jnp.concatenate([x, h], axis=-1)
        gates = jnp.dot(xh, w, preferred_element_type=jnp.float32) + b
        t_i = jnp.tanh(gates[:, 0 * H:1 * H])
        f_g = jax.nn.sigmoid(gates[:, 1 * H:2 * H])
        g_t = jnp.tanh(gates[:, 2 * H:3 * H])
        t_o = jnp.tanh(gates[:, 3 * H:4 * H])
        c_new = f_g * c + 0.5 * (t_i * g_t + g_t)
        ct = jnp.tanh(c_new)
        h_new = 0.5 * (t_o * ct + ct)
        return h_new, c_new

    hfc, cfc = hf[...], cf[...]
    hbc, cbc = hb[...], cb[...]
    for k in range(8):
        hfc, cfc = step(xf_ref[k], hfc, cfc, wf_ref[...], bf_ref[...])
        hbc, cbc = step(xb_ref[7 - k], hbc, cbc, wr_ref[...], br_ref[...])
    hf_new, cf_new, hb_new, cb_new = hfc, cfc, hbc, cbc
    hf[...] = hf_new
    cf[...] = cf_new
    hb[...] = hb_new
    cb[...] = cb_new

    @pl.when(t == L // 8 - 1)
    def _finish():
        out_ref[...] = (
            jnp.dot(hf_new, wfcf_ref[...], preferred_element_type=jnp.float32)
            + jnp.dot(hb_new, wfcr_ref[...], preferred_element_type=jnp.float32)
            + bfc_ref[0, 0])


_lstm_call = pl.pallas_call(
    _lstm_body,
    grid=(L // 8,),
    in_specs=[
        pl.BlockSpec(memory_space=pl.ANY),
        pl.BlockSpec((8, B, E), lambda t: (t, 0, 0)),
        pl.BlockSpec((8, B, E), lambda t: (L // 8 - 1 - t, 0, 0)),
        pl.BlockSpec((E + H, NG), lambda t: (0, 0)),
        pl.BlockSpec((E + H, NG), lambda t: (0, 0)),
        pl.BlockSpec((1, NG), lambda t: (0, 0)),
        pl.BlockSpec((1, NG), lambda t: (0, 0)),
        pl.BlockSpec((H, E), lambda t: (0, 0)),
        pl.BlockSpec((H, E), lambda t: (0, 0)),
        pl.BlockSpec(memory_space=pltpu.SMEM),
    ],
    out_specs=pl.BlockSpec((B, E), lambda t: (0, 0)),
    out_shape=jax.ShapeDtypeStruct((B, E), jnp.float32),
    scratch_shapes=[pltpu.VMEM((B, H), jnp.float32)] * 4
    + [pltpu.SemaphoreType.DMA],
    compiler_params=pltpu.CompilerParams(
        dimension_semantics=("arbitrary",)),
)


def _prep_w(Wih, Whh, bih, bhh):
    w = jnp.concatenate([Wih.T, Whh.T], axis=0)
    b = (bih + bhh).reshape(1, NG)
    scale = jnp.concatenate([
        jnp.full((H,), 0.5, jnp.float32),
        jnp.ones((H,), jnp.float32),
        jnp.ones((H,), jnp.float32),
        jnp.full((H,), 0.5, jnp.float32),
    ])
    return w * scale, b * scale


def kernel(x, emb, Wih_f, Whh_f, bih_f, bhh_f,
           Wih_r, Whh_r, bih_r, bhh_r, W_fc, b_fc):
    idx = jnp.transpose(x).reshape(-1).astype(jnp.int32)
    xs = _make_embed_gather()(emb, idx).reshape(L, B, E)
    wf, bf = _prep_w(Wih_f, Whh_f, bih_f, bhh_f)
    wr, br = _prep_w(Wih_r, Whh_r, bih_r, bhh_r)
    wfcf = jnp.broadcast_to(W_fc[0, :H].reshape(H, 1), (H, E))
    wfcr = jnp.broadcast_to(W_fc[0, H:].reshape(H, 1), (H, E))
    zeros = jnp.zeros((B, H), jnp.float32)
    out = _lstm_call(zeros, xs, xs, wf, wr, bf, br, wfcf, wfcr,
                     b_fc.reshape(1, 1))
    return out[:, 0]


# speedup vs baseline: 1.2837x; 1.0085x over previous
import functools

import jax
import jax.numpy as jnp
from jax import lax
from jax.experimental import pallas as pl
from jax.experimental.pallas import tpu as pltpu
from jax.experimental.pallas import tpu_sc as plsc

VOCAB = 100000
E = 128
H = 128
B = 1024
L = 200
NG = 4 * H

SPLIT_LO = 48
SPLIT_HI = 152
NA = SPLIT_LO + (L - SPLIT_HI)
NB = SPLIT_HI - SPLIT_LO

_NC = 2
_NS = 16
_NW = _NC * _NS
_CH = 256


@functools.cache
def _make_embed_gather(rows):
    per_w = rows // _NW
    nch = per_w // _CH

    @functools.partial(
        pl.kernel,
        mesh=plsc.VectorSubcoreMesh(core_axis_name="c", subcore_axis_name="s"),
        out_type=jax.ShapeDtypeStruct((rows, E), jnp.float32),
        scratch_types=[
            pltpu.VMEM((per_w,), jnp.int32),
            pltpu.VMEM((_CH, E), jnp.float32),
            pltpu.VMEM((_CH, E), jnp.float32),
            pltpu.SemaphoreType.DMA,
            pltpu.SemaphoreType.DMA,
        ],
    )
    def _embed_gather(emb_hbm, idx_hbm, out_hbm, idx_v, rows0, rows1, s0, s1):
        wid = lax.axis_index("s") * _NC + lax.axis_index("c")
        base = wid * per_w
        pltpu.sync_copy(idx_hbm.at[pl.ds(base, per_w)], idx_v)

        def gather(i, rows_v, sem):
            pltpu.async_copy(emb_hbm.at[idx_v.at[pl.ds(i * _CH, _CH)]],
                             rows_v, sem)

        def drain(rows_v, sem):
            pltpu.make_async_copy(emb_hbm.at[pl.ds(0, _CH)], rows_v, sem).wait()

        gather(0, rows0, s0)

        def body(j, carry):
            i = 2 * j
            drain(rows0, s0)
            gather(i + 1, rows1, s1)
            pltpu.sync_copy(rows0, out_hbm.at[pl.ds(base + i * _CH, _CH)])
            drain(rows1, s1)

            @pl.when(i + 2 < nch)
            def _():
                gather(i + 2, rows0, s0)

            pltpu.sync_copy(rows1, out_hbm.at[pl.ds(base + (i + 1) * _CH, _CH)])
            return carry

        lax.fori_loop(0, nch // 2, body, 0)

        if nch % 2:
            i = nch - 1
            drain(rows0, s0)
            pltpu.sync_copy(rows0, out_hbm.at[pl.ds(base + i * _CH, _CH)])

    return _embed_gather


def _lstm_body(state_ref, xf_ref, xb_ref, wf_ref, wr_ref,
               bf_ref, br_ref, wfcf_ref, wfcr_ref, bfc_ref,
               out_ref, st_out, zsem, *, n_iters):
    t = pl.program_id(0)
    hf, cf, hb, cb = (st_out.at[0], st_out.at[1], st_out.at[2], st_out.at[3])

    @pl.when(t == 0)
    def _init():
        for k in range(4):
            pltpu.async_copy(state_ref.at[k], st_out.at[k], zsem)
        for k in range(4):
            pltpu.make_async_copy(state_ref.at[k], st_out.at[k], zsem).wait()

    def step(x, h, c, w, b):
        xh = jnp.concatenate([x, h], axis=-1)
        gates = jnp.dot(xh, w, preferred_element_type=jnp.float32) + b
        t_i = jnp.tanh(gates[:, 0 * H:1 * H])
        f_g = jax.nn.sigmoid(gates[:, 1 * H:2 * H])
        g_t = jnp.tanh(gates[:, 2 * H:3 * H])
        t_o = jnp.tanh(gates[:, 3 * H:4 * H])
        c_new = f_g * c + 0.5 * (t_i * g_t + g_t)
        ct = jnp.tanh(c_new)
        h_new = 0.5 * (t_o * ct + ct)
        return h_new, c_new

    hfc, cfc = hf[...], cf[...]
    hbc, cbc = hb[...], cb[...]
    for k in range(8):
        hfc, cfc = step(xf_ref[k], hfc, cfc, wf_ref[...], bf_ref[...])
        hbc, cbc = step(xb_ref[7 - k], hbc, cbc, wr_ref[...], br_ref[...])
    hf[...] = hfc
    cf[...] = cfc
    hb[...] = hbc
    cb[...] = cbc

    @pl.when(t == n_iters - 1)
    def _finish():
        out_ref[...] = (
            jnp.dot(hfc, wfcf_ref[...], preferred_element_type=jnp.float32)
            + jnp.dot(hbc, wfcr_ref[...], preferred_element_type=jnp.float32)
            + bfc_ref[0, 0])


@functools.cache
def _make_lstm_pass(n_iters, f_off, b_off):
    return pl.pallas_call(
        functools.partial(_lstm_body, n_iters=n_iters),
        grid=(n_iters,),
        in_specs=[
            pl.BlockSpec(memory_space=pl.ANY),
            pl.BlockSpec((8, B, E), lambda t: (f_off + t, 0, 0)),
            pl.BlockSpec((8, B, E), lambda t: (b_off - t, 0, 0)),
            pl.BlockSpec((E + H, NG), lambda t: (0, 0)),
            pl.BlockSpec((E + H, NG), lambda t: (0, 0)),
            pl.BlockSpec((1, NG), lambda t: (0, 0)),
            pl.BlockSpec((1, NG), lambda t: (0, 0)),
            pl.BlockSpec((H, E), lambda t: (0, 0)),
            pl.BlockSpec((H, E), lambda t: (0, 0)),
            pl.BlockSpec(memory_space=pltpu.SMEM),
        ],
        out_specs=[
            pl.BlockSpec((B, E), lambda t: (0, 0)),
            pl.BlockSpec((4, B, H), lambda t: (0, 0, 0)),
        ],
        out_shape=[
            jax.ShapeDtypeStruct((B, E), jnp.float32),
            jax.ShapeDtypeStruct((4, B, H), jnp.float32),
        ],
        scratch_shapes=[pltpu.SemaphoreType.DMA],
        compiler_params=pltpu.CompilerParams(
            dimension_semantics=("arbitrary",)),
    )


def _prep_w(Wih, Whh, bih, bhh):
    w = jnp.concatenate([Wih.T, Whh.T], axis=0)
    b = (bih + bhh).reshape(1, NG)
    scale = jnp.concatenate([
        jnp.full((H,), 0.5, jnp.float32),
        jnp.ones((H,), jnp.float32),
        jnp.ones((H,), jnp.float32),
        jnp.full((H,), 0.5, jnp.float32),
    ])
    return w * scale, b * scale


def kernel(x, emb, Wih_f, Whh_f, bih_f, bhh_f,
           Wih_r, Whh_r, bih_r, bhh_r, W_fc, b_fc):
    idx = jnp.transpose(x).reshape(-1).astype(jnp.int32)
    idx_a = jnp.concatenate([idx[:SPLIT_LO * B], idx[SPLIT_HI * B:]])
    idx_b = idx[SPLIT_LO * B:SPLIT_HI * B]
    xa = _make_embed_gather(NA * B)(emb, idx_a).reshape(NA, B, E)
    xb = _make_embed_gather(NB * B)(emb, idx_b).reshape(NB, B, E)

    wf, bf = _prep_w(Wih_f, Whh_f, bih_f, bhh_f)
    wr, br = _prep_w(Wih_r, Whh_r, bih_r, bhh_r)
    wfcf = jnp.broadcast_to(W_fc[0, :H].reshape(H, 1), (H, E))
    wfcr = jnp.broadcast_to(W_fc[0, H:].reshape(H, 1), (H, E))
    bfc = b_fc.reshape(1, 1)
    weights = (wf, wr, bf, br, wfcf, wfcr, bfc)

    st0 = jnp.zeros((4, B, H), jnp.float32)
    _, st1 = _make_lstm_pass(SPLIT_LO // 8, 0, NA // 8 - 1)(st0, xa, xa, *weights)
    _, st2 = _make_lstm_pass(NB // 8, 0, NB // 8 - 1)(st1, xb, xb, *weights)
    out, _ = _make_lstm_pass(SPLIT_LO // 8, SPLIT_LO // 8, SPLIT_LO // 8 - 1)(
        st2, xa, xa, *weights)
    return out[:, 0]
